# Initial kernel scaffold; baseline (speedup 1.0000x reference)
import jax
import jax.numpy as jnp
from jax.experimental import pallas as pl


def kernel(x, edge_index, edge_attr, batch, We1, be1, We2, be2, Watt, batt, Wnode, bnode, gamma, beta, W1, b1, g2, beta2, W2, b2):
    raise NotImplementedError("write your pallas kernel here")


# speedup vs baseline: 1.7657x; 1.7657x over previous
import jax
import jax.numpy as jnp
from jax import lax
from jax.experimental import pallas as pl
from jax.experimental.pallas import tpu as pltpu
from jax.experimental.pallas import tpu_sc as plsc

_EPS = 1e-5

_N = 10000
_E = 320000
_D = 128
_NG = 64

_NC = 2
_NS = 16
_NW = _NC * _NS
_C = 128
_KCH = 79
_EP = _NW * _KCH * _C

_EB = 2048
_NBLK = 2000
_NP = 10240
_RPT = _NP // _NS


def _mesh():
    return plsc.VectorSubcoreMesh(core_axis_name="c", subcore_axis_name="s",
                                  num_cores=_NC, num_subcores=_NS)


def _sc_gather(table, dst):
    def body(table_hbm, dst_hbm, out_hbm, idx_v, rows_v, sem):
        c = lax.axis_index("c")
        s = lax.axis_index("s")
        wid = s * _NC + c
        base = wid * (_KCH * _C)

        def step(i, carry):
            off = pl.multiple_of(base + i * _C, _C)
            pltpu.sync_copy(dst_hbm.at[pl.ds(off, _C)], idx_v)
            pltpu.async_copy(table_hbm.at[idx_v], rows_v, sem).wait()
            pltpu.sync_copy(rows_v, out_hbm.at[pl.ds(off, _C)])
            return carry

        lax.fori_loop(0, _KCH, step, 0)

    f = pl.kernel(
        body,
        out_type=jax.ShapeDtypeStruct((_EP, 2 * _D), jnp.float32),
        mesh=_mesh(),
        scratch_types=[pltpu.VMEM((_C,), jnp.int32),
                       pltpu.VMEM((_C, 2 * _D), jnp.float32),
                       pltpu.SemaphoreType.DMA],
    )
    return f(table, dst)


def _sc_scatter(msg, dst, zeros_nd):
    def body(msg_hbm, dst_hbm, z_hbm, out_hbm, idx_v, mbuf, acc_sh):
        c = lax.axis_index("c")
        s = lax.axis_index("s")
        rows0 = s * _RPT
        pltpu.sync_copy(z_hbm.at[pl.ds(rows0, _RPT)],
                        acc_sh.at[pl.ds(rows0, _RPT)])
        plsc.subcore_barrier()
        wid = s * _NC + c
        base = wid * (_KCH * _C)

        def step(i, carry):
            off = pl.multiple_of(base + i * _C, _C)
            pltpu.sync_copy(dst_hbm.at[pl.ds(off, _C)], idx_v)
            pltpu.sync_copy(msg_hbm.at[pl.ds(off, _C)], mbuf)
            pltpu.sync_copy(mbuf, acc_sh.at[idx_v], add=True)
            return carry

        lax.fori_loop(0, _KCH, step, 0)
        plsc.subcore_barrier()
        pltpu.sync_copy(acc_sh.at[pl.ds(rows0, _RPT)],
                        out_hbm.at[pl.ds(c * _NP + rows0, _RPT)])

    f = pl.kernel(
        body,
        out_type=jax.ShapeDtypeStruct((2 * _NP, _D), jnp.float32),
        mesh=_mesh(),
        scratch_types=[pltpu.VMEM((_C,), jnp.int32),
                       pltpu.VMEM((_C, _D), jnp.float32),
                       pltpu.VMEM_SHARED((_NP, _D), jnp.float32)],
    )
    return f(msg, dst, zeros_nd)


def _hx(h, Wa, ba, Wn, bn):
    def body(h_ref, wa_ref, ba_ref, wn_ref, bn_ref, out_ref):
        hblk = h_ref[...]
        out_ref[:, :_D] = (jnp.dot(hblk, wa_ref[...],
                                   preferred_element_type=jnp.float32)
                           + ba_ref[...])
        out_ref[:, _D:] = (jnp.dot(hblk, wn_ref[...],
                                   preferred_element_type=jnp.float32)
                           + bn_ref[...])

    return pl.pallas_call(
        body,
        grid=(_N // _NBLK,),
        in_specs=[pl.BlockSpec((_NBLK, _D), lambda i: (i, 0)),
                  pl.BlockSpec((_D, _D), lambda i: (0, 0)),
                  pl.BlockSpec((1, _D), lambda i: (0, 0)),
                  pl.BlockSpec((_D, _D), lambda i: (0, 0)),
                  pl.BlockSpec((1, _D), lambda i: (0, 0))],
        out_specs=pl.BlockSpec((_NBLK, 2 * _D), lambda i: (i, 0)),
        out_shape=jax.ShapeDtypeStruct((_N, 2 * _D), jnp.float32),
    )(h, Wa, ba, Wn, bn)


def _msg(rows, eattr, We1, be1, We2, be2, Wae, Wne):
    def body(r_ref, ea_ref, w1_ref, b1_ref, w2_ref, b2_ref, wa_ref, wn_ref,
             out_ref):
        i = pl.program_id(0)
        e1 = (jnp.dot(ea_ref[...], w1_ref[...],
                      preferred_element_type=jnp.float32) + b1_ref[...])
        ea = (jnp.dot(e1, w2_ref[...],
                      preferred_element_type=jnp.float32) + b2_ref[...])
        a = r_ref[:, :_D] + jnp.dot(ea, wa_ref[...],
                                    preferred_element_type=jnp.float32)
        nn = r_ref[:, _D:] + jnp.dot(ea, wn_ref[...],
                                     preferred_element_type=jnp.float32)
        m = jax.nn.sigmoid(a) * jax.nn.softplus(nn)
        ge = i * _EB + lax.broadcasted_iota(jnp.int32, (_EB, 1), 0)
        out_ref[...] = jnp.where(ge < _E, m, 0.0)

    de = eattr.shape[1]
    dh = We1.shape[1]
    dm = We2.shape[1]
    return pl.pallas_call(
        body,
        grid=(_EP // _EB,),
        in_specs=[pl.BlockSpec((_EB, 2 * _D), lambda i: (i, 0)),
                  pl.BlockSpec((_EB, de), lambda i: (i, 0)),
                  pl.BlockSpec((de, dh), lambda i: (0, 0)),
                  pl.BlockSpec((1, dh), lambda i: (0, 0)),
                  pl.BlockSpec((dh, dm), lambda i: (0, 0)),
                  pl.BlockSpec((1, dm), lambda i: (0, 0)),
                  pl.BlockSpec((dm, _D), lambda i: (0, 0)),
                  pl.BlockSpec((dm, _D), lambda i: (0, 0))],
        out_specs=pl.BlockSpec((_EB, _D), lambda i: (i, 0)),
        out_shape=jax.ShapeDtypeStruct((_EP, _D), jnp.float32),
    )(rows, eattr, We1, be1, We2, be2, Wae, Wne)


def _stats(p0, p1):
    def body(p0_ref, p1_ref, agg_ref, st_ref):
        i = pl.program_id(0)
        s = p0_ref[...] + p1_ref[...]
        agg_ref[...] = s

        @pl.when(i == 0)
        def _():
            st_ref[...] = jnp.zeros_like(st_ref)

        st_ref[0:1, :] += jnp.sum(s, axis=0, keepdims=True)
        st_ref[1:2, :] += jnp.sum(s * s, axis=0, keepdims=True)

    return pl.pallas_call(
        body,
        grid=(_N // _NBLK,),
        in_specs=[pl.BlockSpec((_NBLK, _D), lambda i: (i, 0)),
                  pl.BlockSpec((_NBLK, _D), lambda i: (i, 0))],
        out_specs=[pl.BlockSpec((_NBLK, _D), lambda i: (i, 0)),
                   pl.BlockSpec((8, _D), lambda i: (0, 0))],
        out_shape=[jax.ShapeDtypeStruct((_N, _D), jnp.float32),
                   jax.ShapeDtypeStruct((8, _D), jnp.float32)],
    )(p0, p1)


def _apply(agg, st, gamma, beta, h):
    def body(a_ref, st_ref, g_ref, b_ref, h_ref, out_ref):
        mu = st_ref[0:1, :] / _N
        var = st_ref[1:2, :] / _N - mu * mu
        out_ref[...] = (g_ref[...] * (a_ref[...] - mu)
                        / jnp.sqrt(var + _EPS) + b_ref[...] + h_ref[...])

    return pl.pallas_call(
        body,
        grid=(_N // _NBLK,),
        in_specs=[pl.BlockSpec((_NBLK, _D), lambda i: (i, 0)),
                  pl.BlockSpec((8, _D), lambda i: (0, 0)),
                  pl.BlockSpec((1, _D), lambda i: (0, 0)),
                  pl.BlockSpec((1, _D), lambda i: (0, 0)),
                  pl.BlockSpec((_NBLK, _D), lambda i: (i, 0))],
        out_specs=pl.BlockSpec((_NBLK, _D), lambda i: (i, 0)),
        out_shape=jax.ShapeDtypeStruct((_N, _D), jnp.float32),
    )(agg, st, gamma, beta, h)


def _pool(h, bcol):
    def body(h_ref, b_ref, out_ref, acc_ref):
        i = pl.program_id(0)

        @pl.when(i == 0)
        def _():
            acc_ref[...] = jnp.full_like(acc_ref, -jnp.inf)

        hblk = h_ref[...]
        b = b_ref[...]
        lo = b_ref[0, 0]
        hi = b_ref[_NBLK - 1, 0]

        def gstep(g, carry):
            mask = b == g
            bm = jnp.max(jnp.where(mask, hblk, -jnp.inf), axis=0,
                         keepdims=True)
            acc_ref[pl.ds(g, 1), :] = jnp.maximum(acc_ref[pl.ds(g, 1), :], bm)
            return carry

        lax.fori_loop(lo, hi + 1, gstep, 0)
        out_ref[...] = acc_ref[...]

    return pl.pallas_call(
        body,
        grid=(_N // _NBLK,),
        in_specs=[pl.BlockSpec((_NBLK, _D), lambda i: (i, 0)),
                  pl.BlockSpec((_NBLK, 1), lambda i: (i, 0))],
        out_specs=pl.BlockSpec((_NG, _D), lambda i: (0, 0)),
        out_shape=jax.ShapeDtypeStruct((_NG, _D), jnp.float32),
        scratch_shapes=[pltpu.VMEM((_NG, _D), jnp.float32)],
    )(h, bcol)


def _head(pooled, W1, b1, g2, beta2, W2p, b2p):
    def body(p_ref, w1_ref, b1_ref, g_ref, be_ref, w2_ref, b2_ref, out_ref):
        d = (jnp.dot(p_ref[...], w1_ref[...],
                     preferred_element_type=jnp.float32) + b1_ref[...])
        d = jnp.maximum(d, 0.0)
        mu = jnp.mean(d, axis=0, keepdims=True)
        var = jnp.mean(d * d, axis=0, keepdims=True) - mu * mu
        dn = g_ref[...] * (d - mu) / jnp.sqrt(var + _EPS) + be_ref[...]
        o = (jnp.dot(dn, w2_ref[...],
                     preferred_element_type=jnp.float32) + b2_ref[...])
        out_ref[...] = jax.nn.sigmoid(o)

    dd = W1.shape[1]
    return pl.pallas_call(
        body,
        in_specs=[pl.BlockSpec((_NG, _D), lambda: (0, 0)),
                  pl.BlockSpec((_D, dd), lambda: (0, 0)),
                  pl.BlockSpec((1, dd), lambda: (0, 0)),
                  pl.BlockSpec((1, dd), lambda: (0, 0)),
                  pl.BlockSpec((1, dd), lambda: (0, 0)),
                  pl.BlockSpec((dd, _D), lambda: (0, 0)),
                  pl.BlockSpec((1, _D), lambda: (0, 0))],
        out_specs=pl.BlockSpec((_NG, _D), lambda: (0, 0)),
        out_shape=jax.ShapeDtypeStruct((_NG, _D), jnp.float32),
    )(pooled, W1, b1, g2, beta2, W2p, b2p)


def kernel(x, edge_index, edge_attr, batch, We1, be1, We2, be2, Watt, batt,
           Wnode, bnode, gamma, beta, W1, b1, g2, beta2, W2, b2):
    f32 = jnp.float32
    n_layers = Watt.shape[0]
    out_ch = W2.shape[1]

    dst = edge_index[1].astype(jnp.int32)
    dstp = jnp.zeros((_EP,), jnp.int32).at[:_E].set(dst)
    eap = jnp.zeros((_EP, edge_attr.shape[1]), f32).at[:_E, :].set(edge_attr)
    zeros_nd = jnp.zeros((_NP, _D), f32)
    bcol = batch.astype(jnp.int32).reshape(_N, 1)

    be1r = be1.reshape(1, -1)
    be2r = be2.reshape(1, -1)
    b1r = b1.reshape(1, -1)
    g2r = g2.reshape(1, -1)
    beta2r = beta2.reshape(1, -1)
    W2p = jnp.zeros((W2.shape[0], _D), f32).at[:, :out_ch].set(W2)
    b2p = jnp.zeros((1, _D), f32).at[0, :out_ch].set(b2)

    h = x
    for l in range(n_layers):
        table = _hx(h, Watt[l][:_D], batt[l].reshape(1, _D),
                    Wnode[l][:_D], bnode[l].reshape(1, _D))
        rows = _sc_gather(table, dstp)
        msg = _msg(rows, eap, We1, be1r, We2, be2r,
                   Watt[l][_D:], Wnode[l][_D:])
        part = _sc_scatter(msg, dstp, zeros_nd)
        agg, st = _stats(part[:_N], part[_NP:_NP + _N])
        h = _apply(agg, st, gamma[l].reshape(1, _D),
                   beta[l].reshape(1, _D), h)

    pooled = _pool(h, bcol)
    out = _head(pooled, W1, b1r, g2r, beta2r, W2p, b2p)
    return out[:, :out_ch]


# speedup vs baseline: 1.8551x; 1.0506x over previous
import jax
import jax.numpy as jnp
from jax import lax
from jax.experimental import pallas as pl
from jax.experimental.pallas import tpu as pltpu
from jax.experimental.pallas import tpu_sc as plsc

_EPS = 1e-5

_N = 10000
_E = 320000
_D = 128
_NG = 64

_NC = 2
_NS = 16
_NW = _NC * _NS
_C = 128
_KCH = 80
_EP = _NW * _KCH * _C

_EB = 2048
_NBLK = 2000
_NP = 10240
_RPT = _NP // _NS


def _mesh():
    return plsc.VectorSubcoreMesh(core_axis_name="c", subcore_axis_name="s",
                                  num_cores=_NC, num_subcores=_NS)


def _sc_gather(table, dst):
    def body(table_hbm, dst_hbm, out_hbm, idx_a, idx_b, rows_a, rows_b,
             sia, sib, sga, sgb, ssa, ssb):
        c = lax.axis_index("c")
        s = lax.axis_index("s")
        wid = s * _NC + c
        base = wid * (_KCH * _C)

        def step(i, carry):
            off_a = pl.multiple_of(base + (2 * i) * _C, _C)
            off_b = pl.multiple_of(base + (2 * i + 1) * _C, _C)
            ca = pltpu.async_copy(dst_hbm.at[pl.ds(off_a, _C)], idx_a, sia)
            cb = pltpu.async_copy(dst_hbm.at[pl.ds(off_b, _C)], idx_b, sib)
            ca.wait()
            ga = pltpu.async_copy(table_hbm.at[idx_a], rows_a, sga)
            cb.wait()
            gb = pltpu.async_copy(table_hbm.at[idx_b], rows_b, sgb)
            ga.wait()
            sa = pltpu.async_copy(rows_a, out_hbm.at[pl.ds(off_a, _C)], ssa)
            gb.wait()
            sb = pltpu.async_copy(rows_b, out_hbm.at[pl.ds(off_b, _C)], ssb)
            sa.wait()
            sb.wait()
            return carry

        lax.fori_loop(0, _KCH // 2, step, 0)

    f = pl.kernel(
        body,
        out_type=jax.ShapeDtypeStruct((_EP, _D), jnp.int32),
        mesh=_mesh(),
        scratch_types=[pltpu.VMEM((_C,), jnp.int32),
                       pltpu.VMEM((_C,), jnp.int32),
                       pltpu.VMEM((_C, _D), jnp.int32),
                       pltpu.VMEM((_C, _D), jnp.int32),
                       pltpu.SemaphoreType.DMA,
                       pltpu.SemaphoreType.DMA,
                       pltpu.SemaphoreType.DMA,
                       pltpu.SemaphoreType.DMA,
                       pltpu.SemaphoreType.DMA,
                       pltpu.SemaphoreType.DMA],
    )
    return f(table, dst)


def _sc_scatter(msg, dst, zeros_nd):
    def body(msg_hbm, dst_hbm, z_hbm, out_hbm, idx_a, idx_b, mbuf_a, mbuf_b,
             acc_sh, sia, sib, sma, smb):
        c = lax.axis_index("c")
        s = lax.axis_index("s")
        rows0 = s * _RPT
        pltpu.sync_copy(z_hbm.at[pl.ds(rows0, _RPT)],
                        acc_sh.at[pl.ds(rows0, _RPT)])
        plsc.subcore_barrier()
        wid = s * _NC + c
        base = wid * (_KCH * _C)

        def step(i, carry):
            off_a = pl.multiple_of(base + (2 * i) * _C, _C)
            off_b = pl.multiple_of(base + (2 * i + 1) * _C, _C)
            ia = pltpu.async_copy(dst_hbm.at[pl.ds(off_a, _C)], idx_a, sia)
            ma = pltpu.async_copy(msg_hbm.at[pl.ds(off_a, _C)], mbuf_a, sma)
            ib = pltpu.async_copy(dst_hbm.at[pl.ds(off_b, _C)], idx_b, sib)
            mb = pltpu.async_copy(msg_hbm.at[pl.ds(off_b, _C)], mbuf_b, smb)
            ia.wait()
            ma.wait()
            pltpu.sync_copy(mbuf_a, acc_sh.at[idx_a], add=True)
            ib.wait()
            mb.wait()
            pltpu.sync_copy(mbuf_b, acc_sh.at[idx_b], add=True)
            return carry

        lax.fori_loop(0, _KCH // 2, step, 0)
        plsc.subcore_barrier()
        pltpu.sync_copy(acc_sh.at[pl.ds(rows0, _RPT)],
                        out_hbm.at[pl.ds(c * _NP + rows0, _RPT)])

    f = pl.kernel(
        body,
        out_type=jax.ShapeDtypeStruct((2 * _NP, _D), jnp.float32),
        mesh=_mesh(),
        scratch_types=[pltpu.VMEM((_C,), jnp.int32),
                       pltpu.VMEM((_C,), jnp.int32),
                       pltpu.VMEM((_C, _D), jnp.float32),
                       pltpu.VMEM((_C, _D), jnp.float32),
                       pltpu.VMEM_SHARED((_NP, _D), jnp.float32),
                       pltpu.SemaphoreType.DMA,
                       pltpu.SemaphoreType.DMA,
                       pltpu.SemaphoreType.DMA,
                       pltpu.SemaphoreType.DMA],
    )
    return f(msg, dst, zeros_nd)


def _rne_bf16_bits(x):
    u = lax.bitcast_convert_type(x, jnp.uint32)
    return u + jnp.uint32(0x7FFF) + ((u >> 16) & jnp.uint32(1))


def _pack2_bf16(a, b):
    pa = _rne_bf16_bits(a) >> 16
    pb = _rne_bf16_bits(b) & jnp.uint32(0xFFFF0000)
    return lax.bitcast_convert_type(pa | pb, jnp.int32)


def _unpack2_bf16(p):
    u = lax.bitcast_convert_type(p, jnp.uint32)
    a = lax.bitcast_convert_type(u << 16, jnp.float32)
    b = lax.bitcast_convert_type(u & jnp.uint32(0xFFFF0000), jnp.float32)
    return a, b


def _hx(h, Wa, ba, Wn, bn):
    def body(h_ref, wa_ref, ba_ref, wn_ref, bn_ref, out_ref):
        hblk = h_ref[...]
        a = (jnp.dot(hblk, wa_ref[...], preferred_element_type=jnp.float32)
             + ba_ref[...])
        b = (jnp.dot(hblk, wn_ref[...], preferred_element_type=jnp.float32)
             + bn_ref[...])
        out_ref[...] = _pack2_bf16(a, b)

    return pl.pallas_call(
        body,
        grid=(_N // _NBLK,),
        in_specs=[pl.BlockSpec((_NBLK, _D), lambda i: (i, 0)),
                  pl.BlockSpec((_D, _D), lambda i: (0, 0)),
                  pl.BlockSpec((1, _D), lambda i: (0, 0)),
                  pl.BlockSpec((_D, _D), lambda i: (0, 0)),
                  pl.BlockSpec((1, _D), lambda i: (0, 0))],
        out_specs=pl.BlockSpec((_NBLK, _D), lambda i: (i, 0)),
        out_shape=jax.ShapeDtypeStruct((_N, _D), jnp.int32),
    )(h, Wa, ba, Wn, bn)


def _msg(rows, eattr, We1, be1, We2, be2, Wae, Wne):
    def body(r_ref, ea_ref, w1_ref, b1_ref, w2_ref, b2_ref, wa_ref, wn_ref,
             out_ref):
        i = pl.program_id(0)
        e1 = (jnp.dot(ea_ref[...], w1_ref[...],
                      preferred_element_type=jnp.float32) + b1_ref[...])
        ea = (jnp.dot(e1, w2_ref[...],
                      preferred_element_type=jnp.float32) + b2_ref[...])
        row_a, row_n = _unpack2_bf16(r_ref[...])
        a = row_a + jnp.dot(ea, wa_ref[...],
                            preferred_element_type=jnp.float32)
        nn = row_n + jnp.dot(ea, wn_ref[...],
                             preferred_element_type=jnp.float32)
        m = jax.nn.sigmoid(a) * jax.nn.softplus(nn)
        ge = i * _EB + lax.broadcasted_iota(jnp.int32, (_EB, 1), 0)
        out_ref[...] = jnp.where(ge < _E, m, 0.0)

    de = eattr.shape[1]
    dh = We1.shape[1]
    dm = We2.shape[1]
    return pl.pallas_call(
        body,
        grid=(_EP // _EB,),
        in_specs=[pl.BlockSpec((_EB, _D), lambda i: (i, 0)),
                  pl.BlockSpec((_EB, de), lambda i: (i, 0)),
                  pl.BlockSpec((de, dh), lambda i: (0, 0)),
                  pl.BlockSpec((1, dh), lambda i: (0, 0)),
                  pl.BlockSpec((dh, dm), lambda i: (0, 0)),
                  pl.BlockSpec((1, dm), lambda i: (0, 0)),
                  pl.BlockSpec((dm, _D), lambda i: (0, 0)),
                  pl.BlockSpec((dm, _D), lambda i: (0, 0))],
        out_specs=pl.BlockSpec((_EB, _D), lambda i: (i, 0)),
        out_shape=jax.ShapeDtypeStruct((_EP, _D), jnp.float32),
    )(rows, eattr, We1, be1, We2, be2, Wae, Wne)


def _stats(p0, p1):
    def body(p0_ref, p1_ref, agg_ref, st_ref):
        i = pl.program_id(0)
        s = p0_ref[...] + p1_ref[...]
        agg_ref[...] = s

        @pl.when(i == 0)
        def _():
            st_ref[...] = jnp.zeros_like(st_ref)

        st_ref[0:1, :] += jnp.sum(s, axis=0, keepdims=True)
        st_ref[1:2, :] += jnp.sum(s * s, axis=0, keepdims=True)

    return pl.pallas_call(
        body,
        grid=(_N // _NBLK,),
        in_specs=[pl.BlockSpec((_NBLK, _D), lambda i: (i, 0)),
                  pl.BlockSpec((_NBLK, _D), lambda i: (i, 0))],
        out_specs=[pl.BlockSpec((_NBLK, _D), lambda i: (i, 0)),
                   pl.BlockSpec((8, _D), lambda i: (0, 0))],
        out_shape=[jax.ShapeDtypeStruct((_N, _D), jnp.float32),
                   jax.ShapeDtypeStruct((8, _D), jnp.float32)],
    )(p0, p1)


def _apply(agg, st, gamma, beta, h):
    def body(a_ref, st_ref, g_ref, b_ref, h_ref, out_ref):
        mu = st_ref[0:1, :] / _N
        var = st_ref[1:2, :] / _N - mu * mu
        out_ref[...] = (g_ref[...] * (a_ref[...] - mu)
                        / jnp.sqrt(var + _EPS) + b_ref[...] + h_ref[...])

    return pl.pallas_call(
        body,
        grid=(_N // _NBLK,),
        in_specs=[pl.BlockSpec((_NBLK, _D), lambda i: (i, 0)),
                  pl.BlockSpec((8, _D), lambda i: (0, 0)),
                  pl.BlockSpec((1, _D), lambda i: (0, 0)),
                  pl.BlockSpec((1, _D), lambda i: (0, 0)),
                  pl.BlockSpec((_NBLK, _D), lambda i: (i, 0))],
        out_specs=pl.BlockSpec((_NBLK, _D), lambda i: (i, 0)),
        out_shape=jax.ShapeDtypeStruct((_N, _D), jnp.float32),
    )(agg, st, gamma, beta, h)


def _pool(h, bcol):
    def body(h_ref, b_ref, out_ref, acc_ref):
        i = pl.program_id(0)

        @pl.when(i == 0)
        def _():
            acc_ref[...] = jnp.full_like(acc_ref, -jnp.inf)

        hblk = h_ref[...]
        b = b_ref[...]
        lo = b_ref[0, 0]
        hi = b_ref[_NBLK - 1, 0]

        def gstep(g, carry):
            mask = b == g
            bm = jnp.max(jnp.where(mask, hblk, -jnp.inf), axis=0,
                         keepdims=True)
            acc_ref[pl.ds(g, 1), :] = jnp.maximum(acc_ref[pl.ds(g, 1), :], bm)
            return carry

        lax.fori_loop(lo, hi + 1, gstep, 0)
        out_ref[...] = acc_ref[...]

    return pl.pallas_call(
        body,
        grid=(_N // _NBLK,),
        in_specs=[pl.BlockSpec((_NBLK, _D), lambda i: (i, 0)),
                  pl.BlockSpec((_NBLK, 1), lambda i: (i, 0))],
        out_specs=pl.BlockSpec((_NG, _D), lambda i: (0, 0)),
        out_shape=jax.ShapeDtypeStruct((_NG, _D), jnp.float32),
        scratch_shapes=[pltpu.VMEM((_NG, _D), jnp.float32)],
    )(h, bcol)


def _head(pooled, W1, b1, g2, beta2, W2p, b2p):
    def body(p_ref, w1_ref, b1_ref, g_ref, be_ref, w2_ref, b2_ref, out_ref):
        d = (jnp.dot(p_ref[...], w1_ref[...],
                     preferred_element_type=jnp.float32) + b1_ref[...])
        d = jnp.maximum(d, 0.0)
        mu = jnp.mean(d, axis=0, keepdims=True)
        var = jnp.mean(d * d, axis=0, keepdims=True) - mu * mu
        dn = g_ref[...] * (d - mu) / jnp.sqrt(var + _EPS) + be_ref[...]
        o = (jnp.dot(dn, w2_ref[...],
                     preferred_element_type=jnp.float32) + b2_ref[...])
        out_ref[...] = jax.nn.sigmoid(o)

    dd = W1.shape[1]
    return pl.pallas_call(
        body,
        in_specs=[pl.BlockSpec((_NG, _D), lambda: (0, 0)),
                  pl.BlockSpec((_D, dd), lambda: (0, 0)),
                  pl.BlockSpec((1, dd), lambda: (0, 0)),
                  pl.BlockSpec((1, dd), lambda: (0, 0)),
                  pl.BlockSpec((1, dd), lambda: (0, 0)),
                  pl.BlockSpec((dd, _D), lambda: (0, 0)),
                  pl.BlockSpec((1, _D), lambda: (0, 0))],
        out_specs=pl.BlockSpec((_NG, _D), lambda: (0, 0)),
        out_shape=jax.ShapeDtypeStruct((_NG, _D), jnp.float32),
    )(pooled, W1, b1, g2, beta2, W2p, b2p)


def kernel(x, edge_index, edge_attr, batch, We1, be1, We2, be2, Watt, batt,
           Wnode, bnode, gamma, beta, W1, b1, g2, beta2, W2, b2):
    f32 = jnp.float32
    n_layers = Watt.shape[0]
    out_ch = W2.shape[1]

    dst = edge_index[1].astype(jnp.int32)
    dstp = jnp.zeros((_EP,), jnp.int32).at[:_E].set(dst)
    eap = jnp.zeros((_EP, edge_attr.shape[1]), f32).at[:_E, :].set(edge_attr)
    zeros_nd = jnp.zeros((_NP, _D), f32)
    bcol = batch.astype(jnp.int32).reshape(_N, 1)

    be1r = be1.reshape(1, -1)
    be2r = be2.reshape(1, -1)
    b1r = b1.reshape(1, -1)
    g2r = g2.reshape(1, -1)
    beta2r = beta2.reshape(1, -1)
    W2p = jnp.zeros((W2.shape[0], _D), f32).at[:, :out_ch].set(W2)
    b2p = jnp.zeros((1, _D), f32).at[0, :out_ch].set(b2)

    h = x
    for l in range(n_layers):
        table = _hx(h, Watt[l][:_D], batt[l].reshape(1, _D),
                    Wnode[l][:_D], bnode[l].reshape(1, _D))
        rows = _sc_gather(table, dstp)
        msg = _msg(rows, eap, We1, be1r, We2, be2r,
                   Watt[l][_D:], Wnode[l][_D:])
        part = _sc_scatter(msg, dstp, zeros_nd)
        agg, st = _stats(part[:_N], part[_NP:_NP + _N])
        h = _apply(agg, st, gamma[l].reshape(1, _D),
                   beta[l].reshape(1, _D), h)

    pooled = _pool(h, bcol)
    out = _head(pooled, W1, b1r, g2r, beta2r, W2p, b2p)
    return out[:, :out_ch]


# speedup vs baseline: 3.0699x; 1.6549x over previous
import jax
import jax.numpy as jnp
from jax import lax
from jax.experimental import pallas as pl
from jax.experimental.pallas import tpu as pltpu
from jax.experimental.pallas import tpu_sc as plsc

_EPS = 1e-5

_N = 10000
_E = 320000
_D = 128
_NG = 64

_NC = 2
_NS = 16
_NW = _NC * _NS
_C = 128
_KCH = 80
_EP = _NW * _KCH * _C

_EB = 2048
_NBLK = 2000
_NP = 10240
_RPT = _NP // _NS


def _mesh():
    return plsc.VectorSubcoreMesh(core_axis_name="c", subcore_axis_name="s",
                                  num_cores=_NC, num_subcores=_NS)


def _sc_gather(table, dst):
    def body(table_hbm, dst_hbm, out_hbm, idx_a, idx_b, rows_a, rows_b,
             tab_sh, sia, sib, sga, sgb, ssa, ssb):
        c = lax.axis_index("c")
        s = lax.axis_index("s")
        pltpu.sync_copy(table_hbm.at[pl.ds(s * _RPT, _RPT)],
                        tab_sh.at[pl.ds(s * _RPT, _RPT)])
        plsc.subcore_barrier()
        wid = s * _NC + c
        base = wid * (_KCH * _C)

        def step(i, carry):
            off_a = pl.multiple_of(base + (2 * i) * _C, _C)
            off_b = pl.multiple_of(base + (2 * i + 1) * _C, _C)
            ca = pltpu.async_copy(dst_hbm.at[pl.ds(off_a, _C)], idx_a, sia)
            cb = pltpu.async_copy(dst_hbm.at[pl.ds(off_b, _C)], idx_b, sib)
            ca.wait()
            ga = pltpu.async_copy(tab_sh.at[idx_a], rows_a, sga)
            cb.wait()
            gb = pltpu.async_copy(tab_sh.at[idx_b], rows_b, sgb)
            ga.wait()
            sa = pltpu.async_copy(rows_a, out_hbm.at[pl.ds(off_a, _C)], ssa)
            gb.wait()
            sb = pltpu.async_copy(rows_b, out_hbm.at[pl.ds(off_b, _C)], ssb)
            sa.wait()
            sb.wait()
            return carry

        lax.fori_loop(0, _KCH // 2, step, 0)

    f = pl.kernel(
        body,
        out_type=jax.ShapeDtypeStruct((_EP, _D), jnp.int32),
        mesh=_mesh(),
        scratch_types=[pltpu.VMEM((_C,), jnp.int32),
                       pltpu.VMEM((_C,), jnp.int32),
                       pltpu.VMEM((_C, _D), jnp.int32),
                       pltpu.VMEM((_C, _D), jnp.int32),
                       pltpu.VMEM_SHARED((_NP, _D), jnp.int32),
                       pltpu.SemaphoreType.DMA,
                       pltpu.SemaphoreType.DMA,
                       pltpu.SemaphoreType.DMA,
                       pltpu.SemaphoreType.DMA,
                       pltpu.SemaphoreType.DMA,
                       pltpu.SemaphoreType.DMA],
    )
    return f(table, dst)


def _sc_scatter(msg, dst, zeros_nd):
    def body(msg_hbm, dst_hbm, z_hbm, out_hbm, idx_a, idx_b, mbuf_a, mbuf_b,
             acc_sh, sia, sib, sma, smb):
        c = lax.axis_index("c")
        s = lax.axis_index("s")
        rows0 = s * _RPT
        pltpu.sync_copy(z_hbm.at[pl.ds(rows0, _RPT)],
                        acc_sh.at[pl.ds(rows0, _RPT)])
        plsc.subcore_barrier()
        wid = s * _NC + c
        base = wid * (_KCH * _C)

        def step(i, carry):
            off_a = pl.multiple_of(base + (2 * i) * _C, _C)
            off_b = pl.multiple_of(base + (2 * i + 1) * _C, _C)
            ia = pltpu.async_copy(dst_hbm.at[pl.ds(off_a, _C)], idx_a, sia)
            ma = pltpu.async_copy(msg_hbm.at[pl.ds(off_a, _C)], mbuf_a, sma)
            ib = pltpu.async_copy(dst_hbm.at[pl.ds(off_b, _C)], idx_b, sib)
            mb = pltpu.async_copy(msg_hbm.at[pl.ds(off_b, _C)], mbuf_b, smb)
            ia.wait()
            ma.wait()
            pltpu.sync_copy(mbuf_a, acc_sh.at[idx_a], add=True)
            ib.wait()
            mb.wait()
            pltpu.sync_copy(mbuf_b, acc_sh.at[idx_b], add=True)
            return carry

        lax.fori_loop(0, _KCH // 2, step, 0)
        plsc.subcore_barrier()
        pltpu.sync_copy(acc_sh.at[pl.ds(rows0, _RPT)],
                        out_hbm.at[pl.ds(c * _NP + rows0, _RPT)])

    f = pl.kernel(
        body,
        out_type=jax.ShapeDtypeStruct((2 * _NP, _D), jnp.float32),
        mesh=_mesh(),
        scratch_types=[pltpu.VMEM((_C,), jnp.int32),
                       pltpu.VMEM((_C,), jnp.int32),
                       pltpu.VMEM((_C, _D), jnp.float32),
                       pltpu.VMEM((_C, _D), jnp.float32),
                       pltpu.VMEM_SHARED((_NP, _D), jnp.float32),
                       pltpu.SemaphoreType.DMA,
                       pltpu.SemaphoreType.DMA,
                       pltpu.SemaphoreType.DMA,
                       pltpu.SemaphoreType.DMA],
    )
    return f(msg, dst, zeros_nd)


def _rne_bf16_bits(x):
    u = lax.bitcast_convert_type(x, jnp.uint32)
    return u + jnp.uint32(0x7FFF) + ((u >> 16) & jnp.uint32(1))


def _pack2_bf16(a, b):
    pa = _rne_bf16_bits(a) >> 16
    pb = _rne_bf16_bits(b) & jnp.uint32(0xFFFF0000)
    return lax.bitcast_convert_type(pa | pb, jnp.int32)


def _unpack2_bf16(p):
    u = lax.bitcast_convert_type(p, jnp.uint32)
    a = lax.bitcast_convert_type(u << 16, jnp.float32)
    b = lax.bitcast_convert_type(u & jnp.uint32(0xFFFF0000), jnp.float32)
    return a, b


def _hx(h, Wa, ba, Wn, bn):
    def body(h_ref, wa_ref, ba_ref, wn_ref, bn_ref, out_ref):
        hblk = h_ref[...]
        a = (jnp.dot(hblk, wa_ref[...], preferred_element_type=jnp.float32)
             + ba_ref[...])
        b = (jnp.dot(hblk, wn_ref[...], preferred_element_type=jnp.float32)
             + bn_ref[...])
        out_ref[...] = _pack2_bf16(a, b)

    blk = _NP // 5
    return pl.pallas_call(
        body,
        grid=(5,),
        in_specs=[pl.BlockSpec((blk, _D), lambda i: (i, 0)),
                  pl.BlockSpec((_D, _D), lambda i: (0, 0)),
                  pl.BlockSpec((1, _D), lambda i: (0, 0)),
                  pl.BlockSpec((_D, _D), lambda i: (0, 0)),
                  pl.BlockSpec((1, _D), lambda i: (0, 0))],
        out_specs=pl.BlockSpec((blk, _D), lambda i: (i, 0)),
        out_shape=jax.ShapeDtypeStruct((_NP, _D), jnp.int32),
    )(h, Wa, ba, Wn, bn)


def _msg(rows, eattr, We1, be1, We2, be2, Wae, Wne):
    def body(r_ref, ea_ref, w1_ref, b1_ref, w2_ref, b2_ref, wa_ref, wn_ref,
             out_ref):
        i = pl.program_id(0)
        e1 = (jnp.dot(ea_ref[...], w1_ref[...],
                      preferred_element_type=jnp.float32) + b1_ref[...])
        ea = (jnp.dot(e1, w2_ref[...],
                      preferred_element_type=jnp.float32) + b2_ref[...])
        row_a, row_n = _unpack2_bf16(r_ref[...])
        a = row_a + jnp.dot(ea, wa_ref[...],
                            preferred_element_type=jnp.float32)
        nn = row_n + jnp.dot(ea, wn_ref[...],
                             preferred_element_type=jnp.float32)
        m = jax.nn.sigmoid(a) * jax.nn.softplus(nn)
        ge = i * _EB + lax.broadcasted_iota(jnp.int32, (_EB, 1), 0)
        out_ref[...] = jnp.where(ge < _E, m, 0.0)

    de = eattr.shape[1]
    dh = We1.shape[1]
    dm = We2.shape[1]
    return pl.pallas_call(
        body,
        grid=(_EP // _EB,),
        in_specs=[pl.BlockSpec((_EB, _D), lambda i: (i, 0)),
                  pl.BlockSpec((_EB, de), lambda i: (i, 0)),
                  pl.BlockSpec((de, dh), lambda i: (0, 0)),
                  pl.BlockSpec((1, dh), lambda i: (0, 0)),
                  pl.BlockSpec((dh, dm), lambda i: (0, 0)),
                  pl.BlockSpec((1, dm), lambda i: (0, 0)),
                  pl.BlockSpec((dm, _D), lambda i: (0, 0)),
                  pl.BlockSpec((dm, _D), lambda i: (0, 0))],
        out_specs=pl.BlockSpec((_EB, _D), lambda i: (i, 0)),
        out_shape=jax.ShapeDtypeStruct((_EP, _D), jnp.float32),
    )(rows, eattr, We1, be1, We2, be2, Wae, Wne)


def _stats(p0, p1):
    def body(p0_ref, p1_ref, agg_ref, st_ref):
        i = pl.program_id(0)
        s = p0_ref[...] + p1_ref[...]
        agg_ref[...] = s

        @pl.when(i == 0)
        def _():
            st_ref[...] = jnp.zeros_like(st_ref)

        st_ref[0:1, :] += jnp.sum(s, axis=0, keepdims=True)
        st_ref[1:2, :] += jnp.sum(s * s, axis=0, keepdims=True)

    return pl.pallas_call(
        body,
        grid=(_N // _NBLK,),
        in_specs=[pl.BlockSpec((_NBLK, _D), lambda i: (i, 0)),
                  pl.BlockSpec((_NBLK, _D), lambda i: (i, 0))],
        out_specs=[pl.BlockSpec((_NBLK, _D), lambda i: (i, 0)),
                   pl.BlockSpec((8, _D), lambda i: (0, 0))],
        out_shape=[jax.ShapeDtypeStruct((_N, _D), jnp.float32),
                   jax.ShapeDtypeStruct((8, _D), jnp.float32)],
    )(p0, p1)


def _apply(agg, st, gamma, beta, h):
    def body(a_ref, st_ref, g_ref, b_ref, h_ref, out_ref):
        mu = st_ref[0:1, :] / _N
        var = st_ref[1:2, :] / _N - mu * mu
        out_ref[...] = (g_ref[...] * (a_ref[...] - mu)
                        / jnp.sqrt(var + _EPS) + b_ref[...] + h_ref[...])

    return pl.pallas_call(
        body,
        grid=(_N // _NBLK,),
        in_specs=[pl.BlockSpec((_NBLK, _D), lambda i: (i, 0)),
                  pl.BlockSpec((8, _D), lambda i: (0, 0)),
                  pl.BlockSpec((1, _D), lambda i: (0, 0)),
                  pl.BlockSpec((1, _D), lambda i: (0, 0)),
                  pl.BlockSpec((_NBLK, _D), lambda i: (i, 0))],
        out_specs=pl.BlockSpec((_NBLK, _D), lambda i: (i, 0)),
        out_shape=jax.ShapeDtypeStruct((_N, _D), jnp.float32),
    )(agg, st, gamma, beta, h)


def _pool(h, bcol):
    def body(h_ref, b_ref, out_ref, acc_ref):
        i = pl.program_id(0)

        @pl.when(i == 0)
        def _():
            acc_ref[...] = jnp.full_like(acc_ref, -jnp.inf)

        hblk = h_ref[...]
        b = b_ref[...]
        lo = b_ref[0, 0]
        hi = b_ref[_NBLK - 1, 0]

        def gstep(g, carry):
            mask = b == g
            bm = jnp.max(jnp.where(mask, hblk, -jnp.inf), axis=0,
                         keepdims=True)
            acc_ref[pl.ds(g, 1), :] = jnp.maximum(acc_ref[pl.ds(g, 1), :], bm)
            return carry

        lax.fori_loop(lo, hi + 1, gstep, 0)
        out_ref[...] = acc_ref[...]

    return pl.pallas_call(
        body,
        grid=(_N // _NBLK,),
        in_specs=[pl.BlockSpec((_NBLK, _D), lambda i: (i, 0)),
                  pl.BlockSpec((_NBLK, 1), lambda i: (i, 0))],
        out_specs=pl.BlockSpec((_NG, _D), lambda i: (0, 0)),
        out_shape=jax.ShapeDtypeStruct((_NG, _D), jnp.float32),
        scratch_shapes=[pltpu.VMEM((_NG, _D), jnp.float32)],
    )(h, bcol)


def _head(pooled, W1, b1, g2, beta2, W2p, b2p):
    def body(p_ref, w1_ref, b1_ref, g_ref, be_ref, w2_ref, b2_ref, out_ref):
        d = (jnp.dot(p_ref[...], w1_ref[...],
                     preferred_element_type=jnp.float32) + b1_ref[...])
        d = jnp.maximum(d, 0.0)
        mu = jnp.mean(d, axis=0, keepdims=True)
        var = jnp.mean(d * d, axis=0, keepdims=True) - mu * mu
        dn = g_ref[...] * (d - mu) / jnp.sqrt(var + _EPS) + be_ref[...]
        o = (jnp.dot(dn, w2_ref[...],
                     preferred_element_type=jnp.float32) + b2_ref[...])
        out_ref[...] = jax.nn.sigmoid(o)

    dd = W1.shape[1]
    return pl.pallas_call(
        body,
        in_specs=[pl.BlockSpec((_NG, _D), lambda: (0, 0)),
                  pl.BlockSpec((_D, dd), lambda: (0, 0)),
                  pl.BlockSpec((1, dd), lambda: (0, 0)),
                  pl.BlockSpec((1, dd), lambda: (0, 0)),
                  pl.BlockSpec((1, dd), lambda: (0, 0)),
                  pl.BlockSpec((dd, _D), lambda: (0, 0)),
                  pl.BlockSpec((1, _D), lambda: (0, 0))],
        out_specs=pl.BlockSpec((_NG, _D), lambda: (0, 0)),
        out_shape=jax.ShapeDtypeStruct((_NG, _D), jnp.float32),
    )(pooled, W1, b1, g2, beta2, W2p, b2p)


def kernel(x, edge_index, edge_attr, batch, We1, be1, We2, be2, Watt, batt,
           Wnode, bnode, gamma, beta, W1, b1, g2, beta2, W2, b2):
    f32 = jnp.float32
    n_layers = Watt.shape[0]
    out_ch = W2.shape[1]

    dst = edge_index[1].astype(jnp.int32)
    dstp = jnp.zeros((_EP,), jnp.int32).at[:_E].set(dst)
    eap = jnp.zeros((_EP, edge_attr.shape[1]), f32).at[:_E, :].set(edge_attr)
    zeros_nd = jnp.zeros((_NP, _D), f32)
    bcol = batch.astype(jnp.int32).reshape(_N, 1)

    be1r = be1.reshape(1, -1)
    be2r = be2.reshape(1, -1)
    b1r = b1.reshape(1, -1)
    g2r = g2.reshape(1, -1)
    beta2r = beta2.reshape(1, -1)
    W2p = jnp.zeros((W2.shape[0], _D), f32).at[:, :out_ch].set(W2)
    b2p = jnp.zeros((1, _D), f32).at[0, :out_ch].set(b2)

    h = x
    for l in range(n_layers):
        table = _hx(h, Watt[l][:_D], batt[l].reshape(1, _D),
                    Wnode[l][:_D], bnode[l].reshape(1, _D))
        rows = _sc_gather(table, dstp)
        msg = _msg(rows, eap, We1, be1r, We2, be2r,
                   Watt[l][_D:], Wnode[l][_D:])
        part = _sc_scatter(msg, dstp, zeros_nd)
        agg, st = _stats(part[:_N], part[_NP:_NP + _N])
        h = _apply(agg, st, gamma[l].reshape(1, _D),
                   beta[l].reshape(1, _D), h)

    pooled = _pool(h, bcol)
    out = _head(pooled, W1, b1r, g2r, beta2r, W2p, b2p)
    return out[:, :out_ch]


# speedup vs baseline: 3.2795x; 1.0683x over previous
import jax
import jax.numpy as jnp
from jax import lax
from jax.experimental import pallas as pl
from jax.experimental.pallas import tpu as pltpu
from jax.experimental.pallas import tpu_sc as plsc

_EPS = 1e-5

_N = 10000
_E = 320000
_D = 128
_NG = 64

_NC = 2
_NS = 16
_NW = _NC * _NS
_C = 128
_KCH = 80
_EP = _NW * _KCH * _C

_EB = 2048
_NBLK = 2000
_NP = 10240
_RPT = _NP // _NS


def _mesh():
    return plsc.VectorSubcoreMesh(core_axis_name="c", subcore_axis_name="s",
                                  num_cores=_NC, num_subcores=_NS)


def _sc_gather(table, dst):
    def body(table_hbm, dst_hbm, out_hbm, idx_a, idx_b, rows_a, rows_b,
             tab_sh, sia, sib, sga, sgb, ssa, ssb):
        c = lax.axis_index("c")
        s = lax.axis_index("s")
        pltpu.sync_copy(table_hbm.at[pl.ds(s * _RPT, _RPT)],
                        tab_sh.at[pl.ds(s * _RPT, _RPT)])
        plsc.subcore_barrier()
        wid = s * _NC + c
        base = wid * (_KCH * _C)

        def step(i, carry):
            off_a = pl.multiple_of(base + (2 * i) * _C, _C)
            off_b = pl.multiple_of(base + (2 * i + 1) * _C, _C)
            ca = pltpu.async_copy(dst_hbm.at[pl.ds(off_a, _C)], idx_a, sia)
            cb = pltpu.async_copy(dst_hbm.at[pl.ds(off_b, _C)], idx_b, sib)
            ca.wait()
            ga = pltpu.async_copy(tab_sh.at[idx_a], rows_a, sga)
            cb.wait()
            gb = pltpu.async_copy(tab_sh.at[idx_b], rows_b, sgb)
            ga.wait()
            sa = pltpu.async_copy(rows_a, out_hbm.at[pl.ds(off_a, _C)], ssa)
            gb.wait()
            sb = pltpu.async_copy(rows_b, out_hbm.at[pl.ds(off_b, _C)], ssb)
            sa.wait()
            sb.wait()
            return carry

        lax.fori_loop(0, _KCH // 2, step, 0)

    f = pl.kernel(
        body,
        out_type=jax.ShapeDtypeStruct((_EP, _D), jnp.int32),
        mesh=_mesh(),
        scratch_types=[pltpu.VMEM((_C,), jnp.int32),
                       pltpu.VMEM((_C,), jnp.int32),
                       pltpu.VMEM((_C, _D), jnp.int32),
                       pltpu.VMEM((_C, _D), jnp.int32),
                       pltpu.VMEM_SHARED((_NP, _D), jnp.int32),
                       pltpu.SemaphoreType.DMA,
                       pltpu.SemaphoreType.DMA,
                       pltpu.SemaphoreType.DMA,
                       pltpu.SemaphoreType.DMA,
                       pltpu.SemaphoreType.DMA,
                       pltpu.SemaphoreType.DMA],
    )
    return f(table, dst)


def _sc_scatter(msg, dst, zeros_nd):
    def body(msg_hbm, dst_hbm, z_hbm, out_hbm, idx_a, idx_b, mbuf_a, mbuf_b,
             acc_sh, sia, sib, sma, smb):
        c = lax.axis_index("c")
        s = lax.axis_index("s")
        rows0 = s * _RPT
        pltpu.sync_copy(z_hbm.at[pl.ds(rows0, _RPT)],
                        acc_sh.at[pl.ds(rows0, _RPT)])
        plsc.subcore_barrier()
        wid = s * _NC + c
        base = wid * (_KCH * _C)

        def step(i, carry):
            off_a = pl.multiple_of(base + (2 * i) * _C, _C)
            off_b = pl.multiple_of(base + (2 * i + 1) * _C, _C)
            ia = pltpu.async_copy(dst_hbm.at[pl.ds(off_a, _C)], idx_a, sia)
            ma = pltpu.async_copy(msg_hbm.at[pl.ds(off_a, _C)], mbuf_a, sma)
            ib = pltpu.async_copy(dst_hbm.at[pl.ds(off_b, _C)], idx_b, sib)
            mb = pltpu.async_copy(msg_hbm.at[pl.ds(off_b, _C)], mbuf_b, smb)
            ia.wait()
            ma.wait()
            pltpu.sync_copy(mbuf_a, acc_sh.at[idx_a], add=True)
            ib.wait()
            mb.wait()
            pltpu.sync_copy(mbuf_b, acc_sh.at[idx_b], add=True)
            return carry

        lax.fori_loop(0, _KCH // 2, step, 0)
        plsc.subcore_barrier()
        pltpu.sync_copy(acc_sh.at[pl.ds(rows0, _RPT)],
                        out_hbm.at[pl.ds(c * _NP + rows0, _RPT)])

    f = pl.kernel(
        body,
        out_type=jax.ShapeDtypeStruct((2 * _NP, _D), jnp.float32),
        mesh=_mesh(),
        scratch_types=[pltpu.VMEM((_C,), jnp.int32),
                       pltpu.VMEM((_C,), jnp.int32),
                       pltpu.VMEM((_C, _D), jnp.float32),
                       pltpu.VMEM((_C, _D), jnp.float32),
                       pltpu.VMEM_SHARED((_NP, _D), jnp.float32),
                       pltpu.SemaphoreType.DMA,
                       pltpu.SemaphoreType.DMA,
                       pltpu.SemaphoreType.DMA,
                       pltpu.SemaphoreType.DMA],
    )
    return f(msg, dst, zeros_nd)


def _rne_bf16_bits(x):
    u = lax.bitcast_convert_type(x, jnp.uint32)
    return u + jnp.uint32(0x7FFF) + ((u >> 16) & jnp.uint32(1))


def _pack2_bf16(a, b):
    pa = _rne_bf16_bits(a) >> 16
    pb = _rne_bf16_bits(b) & jnp.uint32(0xFFFF0000)
    return lax.bitcast_convert_type(pa | pb, jnp.int32)


def _unpack2_bf16(p):
    u = lax.bitcast_convert_type(p, jnp.uint32)
    a = lax.bitcast_convert_type(u << 16, jnp.float32)
    b = lax.bitcast_convert_type(u & jnp.uint32(0xFFFF0000), jnp.float32)
    return a, b


def _hx(h, Wa, ba, Wn, bn):
    def body(h_ref, wa_ref, ba_ref, wn_ref, bn_ref, out_ref):
        hblk = h_ref[...]
        a = (jnp.dot(hblk, wa_ref[...], preferred_element_type=jnp.float32)
             + ba_ref[...])
        b = (jnp.dot(hblk, wn_ref[...], preferred_element_type=jnp.float32)
             + bn_ref[...])
        out_ref[...] = _pack2_bf16(a, b)

    blk = _NP // 5
    return pl.pallas_call(
        body,
        grid=(5,),
        in_specs=[pl.BlockSpec((blk, _D), lambda i: (i, 0)),
                  pl.BlockSpec((_D, _D), lambda i: (0, 0)),
                  pl.BlockSpec((1, _D), lambda i: (0, 0)),
                  pl.BlockSpec((_D, _D), lambda i: (0, 0)),
                  pl.BlockSpec((1, _D), lambda i: (0, 0))],
        out_specs=pl.BlockSpec((blk, _D), lambda i: (i, 0)),
        out_shape=jax.ShapeDtypeStruct((_NP, _D), jnp.int32),
    )(h, Wa, ba, Wn, bn)


def _msg(rows, eattr, Wca, bca, Wcn, bcn):
    def body(r_ref, ea_ref, wa_ref, ba_ref, wn_ref, bn_ref, out_ref):
        i = pl.program_id(0)
        eb = ea_ref[...]
        row_a, row_n = _unpack2_bf16(r_ref[...])
        a = row_a + (jnp.dot(eb, wa_ref[...],
                             preferred_element_type=jnp.float32) + ba_ref[...])
        nn = row_n + (jnp.dot(eb, wn_ref[...],
                              preferred_element_type=jnp.float32) + bn_ref[...])
        s = 0.5 * jnp.tanh(a * 0.5) + 0.5
        t = lax.exp2(jnp.abs(nn) * (-1.4426950408889634))
        sp = jnp.maximum(nn, 0.0) + jnp.log2(1.0 + t) * 0.6931471805599453
        m = s * sp
        ge = i * _EB + lax.broadcasted_iota(jnp.int32, (_EB, 1), 0)
        out_ref[...] = jnp.where(ge < _E, m, 0.0)

    de = eattr.shape[1]
    return pl.pallas_call(
        body,
        grid=(_EP // _EB,),
        in_specs=[pl.BlockSpec((_EB, _D), lambda i: (i, 0)),
                  pl.BlockSpec((_EB, de), lambda i: (i, 0)),
                  pl.BlockSpec((de, _D), lambda i: (0, 0)),
                  pl.BlockSpec((1, _D), lambda i: (0, 0)),
                  pl.BlockSpec((de, _D), lambda i: (0, 0)),
                  pl.BlockSpec((1, _D), lambda i: (0, 0))],
        out_specs=pl.BlockSpec((_EB, _D), lambda i: (i, 0)),
        out_shape=jax.ShapeDtypeStruct((_EP, _D), jnp.float32),
    )(rows, eattr, Wca, bca, Wcn, bcn)


def _stats(p0, p1):
    def body(p0_ref, p1_ref, agg_ref, st_ref):
        i = pl.program_id(0)
        s = p0_ref[...] + p1_ref[...]
        agg_ref[...] = s

        @pl.when(i == 0)
        def _():
            st_ref[...] = jnp.zeros_like(st_ref)

        st_ref[0:1, :] += jnp.sum(s, axis=0, keepdims=True)
        st_ref[1:2, :] += jnp.sum(s * s, axis=0, keepdims=True)

    return pl.pallas_call(
        body,
        grid=(_N // _NBLK,),
        in_specs=[pl.BlockSpec((_NBLK, _D), lambda i: (i, 0)),
                  pl.BlockSpec((_NBLK, _D), lambda i: (i, 0))],
        out_specs=[pl.BlockSpec((_NBLK, _D), lambda i: (i, 0)),
                   pl.BlockSpec((8, _D), lambda i: (0, 0))],
        out_shape=[jax.ShapeDtypeStruct((_N, _D), jnp.float32),
                   jax.ShapeDtypeStruct((8, _D), jnp.float32)],
    )(p0, p1)


def _apply(agg, st, gamma, beta, h):
    def body(a_ref, st_ref, g_ref, b_ref, h_ref, out_ref):
        mu = st_ref[0:1, :] / _N
        var = st_ref[1:2, :] / _N - mu * mu
        out_ref[...] = (g_ref[...] * (a_ref[...] - mu)
                        / jnp.sqrt(var + _EPS) + b_ref[...] + h_ref[...])

    return pl.pallas_call(
        body,
        grid=(_N // _NBLK,),
        in_specs=[pl.BlockSpec((_NBLK, _D), lambda i: (i, 0)),
                  pl.BlockSpec((8, _D), lambda i: (0, 0)),
                  pl.BlockSpec((1, _D), lambda i: (0, 0)),
                  pl.BlockSpec((1, _D), lambda i: (0, 0)),
                  pl.BlockSpec((_NBLK, _D), lambda i: (i, 0))],
        out_specs=pl.BlockSpec((_NBLK, _D), lambda i: (i, 0)),
        out_shape=jax.ShapeDtypeStruct((_N, _D), jnp.float32),
    )(agg, st, gamma, beta, h)


def _pool(h, bcol):
    def body(h_ref, b_ref, out_ref, acc_ref):
        i = pl.program_id(0)

        @pl.when(i == 0)
        def _():
            acc_ref[...] = jnp.full_like(acc_ref, -jnp.inf)

        hblk = h_ref[...]
        b = b_ref[...]
        lo = b_ref[0, 0]
        hi = b_ref[_NBLK - 1, 0]

        def gstep(g, carry):
            mask = b == g
            bm = jnp.max(jnp.where(mask, hblk, -jnp.inf), axis=0,
                         keepdims=True)
            acc_ref[pl.ds(g, 1), :] = jnp.maximum(acc_ref[pl.ds(g, 1), :], bm)
            return carry

        lax.fori_loop(lo, hi + 1, gstep, 0)
        out_ref[...] = acc_ref[...]

    return pl.pallas_call(
        body,
        grid=(_N // _NBLK,),
        in_specs=[pl.BlockSpec((_NBLK, _D), lambda i: (i, 0)),
                  pl.BlockSpec((_NBLK, 1), lambda i: (i, 0))],
        out_specs=pl.BlockSpec((_NG, _D), lambda i: (0, 0)),
        out_shape=jax.ShapeDtypeStruct((_NG, _D), jnp.float32),
        scratch_shapes=[pltpu.VMEM((_NG, _D), jnp.float32)],
    )(h, bcol)


def _head(pooled, W1, b1, g2, beta2, W2p, b2p):
    def body(p_ref, w1_ref, b1_ref, g_ref, be_ref, w2_ref, b2_ref, out_ref):
        d = (jnp.dot(p_ref[...], w1_ref[...],
                     preferred_element_type=jnp.float32) + b1_ref[...])
        d = jnp.maximum(d, 0.0)
        mu = jnp.mean(d, axis=0, keepdims=True)
        var = jnp.mean(d * d, axis=0, keepdims=True) - mu * mu
        dn = g_ref[...] * (d - mu) / jnp.sqrt(var + _EPS) + be_ref[...]
        o = (jnp.dot(dn, w2_ref[...],
                     preferred_element_type=jnp.float32) + b2_ref[...])
        out_ref[...] = jax.nn.sigmoid(o)

    dd = W1.shape[1]
    return pl.pallas_call(
        body,
        in_specs=[pl.BlockSpec((_NG, _D), lambda: (0, 0)),
                  pl.BlockSpec((_D, dd), lambda: (0, 0)),
                  pl.BlockSpec((1, dd), lambda: (0, 0)),
                  pl.BlockSpec((1, dd), lambda: (0, 0)),
                  pl.BlockSpec((1, dd), lambda: (0, 0)),
                  pl.BlockSpec((dd, _D), lambda: (0, 0)),
                  pl.BlockSpec((1, _D), lambda: (0, 0))],
        out_specs=pl.BlockSpec((_NG, _D), lambda: (0, 0)),
        out_shape=jax.ShapeDtypeStruct((_NG, _D), jnp.float32),
    )(pooled, W1, b1, g2, beta2, W2p, b2p)


def kernel(x, edge_index, edge_attr, batch, We1, be1, We2, be2, Watt, batt,
           Wnode, bnode, gamma, beta, W1, b1, g2, beta2, W2, b2):
    f32 = jnp.float32
    n_layers = Watt.shape[0]
    out_ch = W2.shape[1]

    dst = edge_index[1].astype(jnp.int32)
    dstp = jnp.zeros((_EP,), jnp.int32).at[:_E].set(dst)
    eap = jnp.zeros((_EP, edge_attr.shape[1]), f32).at[:_E, :].set(edge_attr)
    zeros_nd = jnp.zeros((_NP, _D), f32)
    bcol = batch.astype(jnp.int32).reshape(_N, 1)

    be1r = be1.reshape(1, -1)
    be2r = be2.reshape(1, -1)
    b1r = b1.reshape(1, -1)
    g2r = g2.reshape(1, -1)
    beta2r = beta2.reshape(1, -1)
    W2p = jnp.zeros((W2.shape[0], _D), f32).at[:, :out_ch].set(W2)
    b2p = jnp.zeros((1, _D), f32).at[0, :out_ch].set(b2)

    ea_lin = be1 @ We2 + be2

    h = x
    for l in range(n_layers):
        table = _hx(h, Watt[l][:_D], batt[l].reshape(1, _D),
                    Wnode[l][:_D], bnode[l].reshape(1, _D))
        rows = _sc_gather(table, dstp)
        Wae = Watt[l][_D:]
        Wne = Wnode[l][_D:]
        msg = _msg(rows, eap, We1 @ (We2 @ Wae), (ea_lin @ Wae).reshape(1, _D),
                   We1 @ (We2 @ Wne), (ea_lin @ Wne).reshape(1, _D))
        part = _sc_scatter(msg, dstp, zeros_nd)
        agg, st = _stats(part[:_N], part[_NP:_NP + _N])
        h = _apply(agg, st, gamma[l].reshape(1, _D),
                   beta[l].reshape(1, _D), h)

    pooled = _pool(h, bcol)
    out = _head(pooled, W1, b1r, g2r, beta2r, W2p, b2p)
    return out[:, :out_ch]
